# initial kernel scaffold (unmeasured)
import jax
import jax.numpy as jnp
from jax import lax
from jax.experimental import pallas as pl
from jax.experimental.pallas import tpu as pltpu

N_DEV = 4


def kernel(x, w_mat, scale_x, scale_w):
    m_tot, k_loc = x.shape
    k_tot, n_tot = w_mat.shape
    m_blk = m_tot // N_DEV
    n_half = n_tot // 2
    nc_blk = 1024

    def body(x_ref, w_ref, sx_ref, sw_ref, out_ref,
             xall, wbuf, send_sems, recv_sems, wsems):
        my = lax.axis_index("i")

        order = (0, 1, 3, 2)
        units = [(e, h) for e in order for h in (0, 1)]

        wdescs = {}

        def start_w(i):
            e, h = units[i]
            s = (my + e) % N_DEV
            c = pltpu.make_async_copy(
                w_ref.at[pl.ds(s * k_loc, k_loc), pl.ds(h * n_half, n_half)],
                wbuf.at[i % 2],
                wsems.at[i % 2],
            )
            c.start()
            wdescs[i] = c

        start_w(0)
        start_w(1)

        bar = pltpu.get_barrier_semaphore()
        for d in (1, 2, 3):
            pl.semaphore_signal(
                bar, inc=1,
                device_id=((my + d) % N_DEV,),
                device_id_type=pl.DeviceIdType.MESH,
            )
        pl.semaphore_wait(bar, 3)

        sends = []
        for d in (1, 2, 3):
            peer = (my + d) % N_DEV
            r = pltpu.make_async_remote_copy(
                src_ref=x_ref.at[pl.ds(peer * m_blk, m_blk), :],
                dst_ref=xall.at[3 - d],
                send_sem=send_sems.at[d - 1],
                recv_sem=recv_sems.at[3 - d],
                device_id=(peer,),
                device_id_type=pl.DeviceIdType.MESH,
            )
            r.start()
            sends.append(r)

        scale = sx_ref[0] * sw_ref[0]

        for i, (e, h) in enumerate(units):
            if e != 0 and h == 0:
                pltpu.make_async_remote_copy(
                    src_ref=xall.at[e - 1],
                    dst_ref=xall.at[e - 1],
                    send_sem=send_sems.at[e - 1],
                    recv_sem=recv_sems.at[e - 1],
                    device_id=(my,),
                    device_id_type=pl.DeviceIdType.MESH,
                ).wait_recv()
            wdescs.pop(i).wait()
            if i + 2 < len(units):
                start_w(i + 2)

            if e == 0:
                xb = x_ref[pl.ds(my * m_blk, m_blk), :].astype(jnp.bfloat16)
            else:
                xb = xall[e - 1].astype(jnp.bfloat16)
            for nc in range(n_half // nc_blk):
                csl = pl.ds(h * n_half + nc * nc_blk, nc_blk)
                wb = wbuf[i % 2, :, nc * nc_blk:(nc + 1) * nc_blk].astype(
                    jnp.bfloat16)
                part = jnp.dot(xb, wb, preferred_element_type=jnp.float32)
                if e == 0:
                    out_ref[:, csl] = part
                else:
                    out_ref[:, csl] = out_ref[:, csl] + part

        for r in sends:
            r.wait_send()

        for nc in range(n_tot // nc_blk):
            csl = pl.ds(nc * nc_blk, nc_blk)
            y = out_ref[:, csl] * scale
            out_ref[:, csl] = y * (1.0 / (1.0 + jnp.exp(-y)))

    return pl.pallas_call(
        body,
        out_shape=jax.ShapeDtypeStruct((m_blk, n_tot), jnp.float32),
        in_specs=[
            pl.BlockSpec(memory_space=pltpu.VMEM),
            pl.BlockSpec(memory_space=pltpu.ANY),
            pl.BlockSpec(memory_space=pltpu.SMEM),
            pl.BlockSpec(memory_space=pltpu.SMEM),
        ],
        out_specs=pl.BlockSpec(memory_space=pltpu.VMEM),
        scratch_shapes=[
            pltpu.VMEM((N_DEV - 1, m_blk, k_loc), jnp.int8),
            pltpu.VMEM((2, k_loc, n_half), jnp.int8),
            pltpu.SemaphoreType.DMA((N_DEV - 1,)),
            pltpu.SemaphoreType.DMA((N_DEV - 1,)),
            pltpu.SemaphoreType.DMA((2,)),
        ],
        compiler_params=pltpu.CompilerParams(collective_id=0),
    )(x, w_mat, scale_x, scale_w)


# baseline (device time: 126445 ns/iter reference)
import jax
import jax.numpy as jnp
from jax import lax
from jax.experimental import pallas as pl
from jax.experimental.pallas import tpu as pltpu

N_DEV = 4


def kernel(x, w_mat, scale_x, scale_w):
    m_tot, k_loc = x.shape
    k_tot, n_tot = w_mat.shape
    m_blk = m_tot // N_DEV
    n_half = n_tot // 2
    nc_blk = 1024

    def body(x_ref, w_ref, sx_ref, sw_ref, out_ref,
             xall, wbuf, send_sems, recv_sems, wsems):
        my = lax.axis_index("i")

        order = (0, 1, 3, 2)
        units = [(e, h) for e in order for h in (0, 1)]

        wdescs = {}

        def start_w(i):
            e, h = units[i]
            s = (my + e) % N_DEV
            c = pltpu.make_async_copy(
                w_ref.at[pl.ds(s * k_loc, k_loc), pl.ds(h * n_half, n_half)],
                wbuf.at[i % 2],
                wsems.at[i % 2],
            )
            c.start()
            wdescs[i] = c

        start_w(0)
        start_w(1)

        bar = pltpu.get_barrier_semaphore()
        for d in (1, 2, 3):
            pl.semaphore_signal(
                bar, inc=1,
                device_id=((my + d) % N_DEV,),
                device_id_type=pl.DeviceIdType.MESH,
            )
        pl.semaphore_wait(bar, 3)

        sends = []
        for d in (1, 2, 3):
            peer = (my + d) % N_DEV
            r = pltpu.make_async_remote_copy(
                src_ref=x_ref.at[pl.ds(peer * m_blk, m_blk), :],
                dst_ref=xall.at[3 - d],
                send_sem=send_sems.at[d - 1],
                recv_sem=recv_sems.at[3 - d],
                device_id=(peer,),
                device_id_type=pl.DeviceIdType.MESH,
            )
            r.start()
            sends.append(r)

        scale = sx_ref[0] * sw_ref[0]

        for i, (e, h) in enumerate(units):
            if e != 0 and h == 0:
                pltpu.make_async_remote_copy(
                    src_ref=xall.at[e - 1],
                    dst_ref=xall.at[e - 1],
                    send_sem=send_sems.at[e - 1],
                    recv_sem=recv_sems.at[e - 1],
                    device_id=(my,),
                    device_id_type=pl.DeviceIdType.MESH,
                ).wait_recv()
            wdescs.pop(i).wait()

            if e == 0:
                xb = x_ref[pl.ds(my * m_blk, m_blk), :].astype(jnp.bfloat16)
            else:
                xb = xall[e - 1].astype(jnp.bfloat16)
            for nc in range(n_half // nc_blk):
                csl = pl.ds(h * n_half + nc * nc_blk, nc_blk)
                wb = wbuf[i % 2, :, nc * nc_blk:(nc + 1) * nc_blk].astype(
                    jnp.bfloat16)
                part = jnp.dot(xb, wb, preferred_element_type=jnp.float32)
                if e == 0:
                    out_ref[:, csl] = part
                else:
                    out_ref[:, csl] = out_ref[:, csl] + part
            if i + 2 < len(units):
                start_w(i + 2)

        for r in sends:
            r.wait_send()

        for nc in range(n_tot // nc_blk):
            csl = pl.ds(nc * nc_blk, nc_blk)
            y = out_ref[:, csl] * scale
            out_ref[:, csl] = y * (1.0 / (1.0 + jnp.exp(-y)))

    return pl.pallas_call(
        body,
        out_shape=jax.ShapeDtypeStruct((m_blk, n_tot), jnp.float32),
        in_specs=[
            pl.BlockSpec(memory_space=pltpu.VMEM),
            pl.BlockSpec(memory_space=pl.ANY),
            pl.BlockSpec(memory_space=pltpu.SMEM),
            pl.BlockSpec(memory_space=pltpu.SMEM),
        ],
        out_specs=pl.BlockSpec(memory_space=pltpu.VMEM),
        scratch_shapes=[
            pltpu.VMEM((N_DEV - 1, m_blk, k_loc), jnp.int8),
            pltpu.VMEM((2, k_loc, n_half), jnp.int8),
            pltpu.SemaphoreType.DMA((N_DEV - 1,)),
            pltpu.SemaphoreType.DMA((N_DEV - 1,)),
            pltpu.SemaphoreType.DMA((2,)),
        ],
        compiler_params=pltpu.CompilerParams(
            collective_id=0,
            vmem_limit_bytes=64 * 1024 * 1024,
        ),
    )(x, w_mat, scale_x, scale_w)


# device time: 107220 ns/iter; 1.1793x vs baseline; 1.1793x over previous
import jax
import jax.numpy as jnp
from jax import lax
from jax.experimental import pallas as pl
from jax.experimental.pallas import tpu as pltpu
import contextlib

N_DEV = 4
DEBUG_NO_RDMA = False
DEBUG_SCOPES = False


def _scope(name):
    return jax.named_scope(name) if DEBUG_SCOPES else contextlib.nullcontext()


def kernel(x, w_mat, scale_x, scale_w):
    m_tot, k_loc = x.shape
    k_tot, n_tot = w_mat.shape
    m_blk = m_tot // N_DEV
    n_half = n_tot // 2
    nc_blk = 1024

    def body(x_ref, w_ref, sx_ref, sw_ref, out_ref,
             xall, acc, wbuf, send_sems, recv_sems, wsems, osems):
        my = lax.axis_index("i")

        order = (0, 3, 1, 2)
        units = [(e, h) for e in order for h in (0, 1)]

        wdescs = {}

        def start_w(i):
            e, h = units[i]
            s = (my + e) % N_DEV
            c = pltpu.make_async_copy(
                w_ref.at[pl.ds(s * k_loc, k_loc), pl.ds(h * n_half, n_half)],
                wbuf.at[i % 2],
                wsems.at[i % 2],
            )
            c.start()
            wdescs[i] = c

        start_w(0)
        start_w(1)

        if not DEBUG_NO_RDMA:
            with _scope("barrier"):
                bar = pltpu.get_barrier_semaphore()
                for d in (1, 2, 3):
                    pl.semaphore_signal(
                        bar, inc=1,
                        device_id=((my + d) % N_DEV,),
                        device_id_type=pl.DeviceIdType.MESH,
                    )
                pl.semaphore_wait(bar, 3)

        sends = []
        for d in () if DEBUG_NO_RDMA else (1, 2, 3):
            peer = (my + d) % N_DEV
            r = pltpu.make_async_remote_copy(
                src_ref=x_ref.at[pl.ds(peer * m_blk, m_blk), :],
                dst_ref=xall.at[3 - d],
                send_sem=send_sems.at[d - 1],
                recv_sem=recv_sems.at[3 - d],
                device_id=(peer,),
                device_id_type=pl.DeviceIdType.MESH,
            )
            r.start()
            sends.append(r)

        scale = sx_ref[0] * sw_ref[0]
        odescs = []

        for i, (e, h) in enumerate(units):
            if e != 0 and h == 0 and not DEBUG_NO_RDMA:
                with _scope(f"waitrecv#e={e}"):
                    pltpu.make_async_remote_copy(
                        src_ref=xall.at[e - 1],
                        dst_ref=xall.at[e - 1],
                        send_sem=send_sems.at[e - 1],
                        recv_sem=recv_sems.at[e - 1],
                        device_id=(my,),
                        device_id_type=pl.DeviceIdType.MESH,
                    ).wait_recv()
            with _scope(f"waitw#i={i}"):
                wdescs.pop(i).wait()

            with _scope(f"compute#i={i}"):
                if h == 0:
                    if e == 0 or DEBUG_NO_RDMA:
                        xb = x_ref[pl.ds(my * m_blk, m_blk), :].astype(
                            jnp.bfloat16)
                    else:
                        xb = xall[e - 1].astype(jnp.bfloat16)
                for nc in range(n_half // nc_blk):
                    csl = pl.ds(h * n_half + nc * nc_blk, nc_blk)
                    wb = wbuf[i % 2, :, nc * nc_blk:(nc + 1) * nc_blk].astype(
                        jnp.bfloat16)
                    part = jnp.dot(xb, wb, preferred_element_type=jnp.float32)
                    if e == 0:
                        acc[:, csl] = part
                    elif e == order[-1]:
                        y = (acc[:, csl] + part) * scale
                        acc[:, csl] = y * (1.0 / (1.0 + jnp.exp(-y)))
                        k = h * (n_half // nc_blk) + nc
                        c = pltpu.make_async_copy(
                            acc.at[:, csl], out_ref.at[:, csl], osems.at[k])
                        c.start()
                        odescs.append(c)
                    else:
                        acc[:, csl] = acc[:, csl] + part
                if i + 2 < len(units):
                    start_w(i + 2)

        with _scope("drain"):
            for r in sends:
                r.wait_send()
            for c in odescs:
                c.wait()

    return pl.pallas_call(
        body,
        out_shape=jax.ShapeDtypeStruct((m_blk, n_tot), jnp.float32),
        in_specs=[
            pl.BlockSpec(memory_space=pltpu.VMEM),
            pl.BlockSpec(memory_space=pl.ANY),
            pl.BlockSpec(memory_space=pltpu.SMEM),
            pl.BlockSpec(memory_space=pltpu.SMEM),
        ],
        out_specs=pl.BlockSpec(memory_space=pl.ANY),
        scratch_shapes=[
            pltpu.VMEM((N_DEV - 1, m_blk, k_loc), jnp.int8),
            pltpu.VMEM((m_blk, n_tot), jnp.float32),
            pltpu.VMEM((2, k_loc, n_half), jnp.int8),
            pltpu.SemaphoreType.DMA((N_DEV - 1,)),
            pltpu.SemaphoreType.DMA((N_DEV - 1,)),
            pltpu.SemaphoreType.DMA((2,)),
            pltpu.SemaphoreType.DMA((n_tot // nc_blk,)),
        ],
        compiler_params=pltpu.CompilerParams(
            collective_id=None if DEBUG_NO_RDMA else 0,
            vmem_limit_bytes=64 * 1024 * 1024,
        ),
    )(x, w_mat, scale_x, scale_w)
